# baseline (speedup 1.0000x reference)
import functools

import jax
import jax.numpy as jnp
from jax.experimental import pallas as pl
from jax.experimental.pallas import tpu as pltpu
from jax.experimental.pallas import tpu_sc as plsc


def _enc_body(x_ref, w1_ref, b1_ref, w2_ref, b2_ref, w3_ref, b3_ref, wc_ref,
              ze_ref, idx_ref):
    xb = x_ref[...]
    h = jnp.dot(xb, w1_ref[...], preferred_element_type=jnp.float32) + b1_ref[...]
    h = jnp.maximum(h, 0.0)
    h = jnp.dot(h, w2_ref[...], preferred_element_type=jnp.float32) + b2_ref[...]
    h = jnp.maximum(h, 0.0)
    ze = jnp.dot(h, w3_ref[...], preferred_element_type=jnp.float32) + b3_ref[...]
    ze_ref[...] = ze

    wc = wc_ref[...]
    cnorm = jnp.sum(wc * wc, axis=0, keepdims=True)
    dist = cnorm - 2.0 * jnp.dot(ze, wc, preferred_element_type=jnp.float32)
    idx = jnp.argmin(dist, axis=1).astype(jnp.int32)
    idx_ref[...] = idx.reshape(1, 1, -1)


def _encode_tc(xc, W1, b1r, W2, b2r, W3, b3r, Wc, bb):
    n, IN = xc.shape
    EMB = W3.shape[1]
    grid = (n // bb,)

    def full(a):
        return pl.BlockSpec(a.shape, lambda i: (0,) * a.ndim)

    out_shapes = (
        jax.ShapeDtypeStruct((n, EMB), jnp.float32),
        jax.ShapeDtypeStruct((n // bb, 1, bb), jnp.int32),
    )
    out_specs = (
        pl.BlockSpec((bb, EMB), lambda i: (i, 0)),
        pl.BlockSpec((1, 1, bb), lambda i: (i, 0, 0)),
    )
    in_specs = [pl.BlockSpec((bb, IN), lambda i: (i, 0))] + [
        full(a) for a in (W1, b1r, W2, b2r, W3, b3r, Wc)]
    return pl.pallas_call(
        _enc_body,
        grid=grid,
        in_specs=in_specs,
        out_specs=out_specs,
        out_shape=out_shapes,
    )(xc, W1, b1r, W2, b2r, W3, b3r, Wc)


_NC, _NS = 2, 16
_NW = _NC * _NS
_CH = 128


def _gather_sc(table, idx_flat):
    n = idx_flat.shape[0]
    emb_dim = table.shape[1]
    bpw = n // _NW
    mesh = plsc.VectorSubcoreMesh(core_axis_name="c", subcore_axis_name="s")

    @functools.partial(
        pl.kernel,
        out_type=jax.ShapeDtypeStruct((n, emb_dim), jnp.float32),
        mesh=mesh,
        scratch_types=[
            pltpu.VMEM((_CH,), jnp.int32),
            pltpu.VMEM((_CH, emb_dim), jnp.float32),
            pltpu.SemaphoreType.DMA,
        ])
    def k(tab_hbm, i_hbm, o_hbm, idx_v, rows_v, sem):
        wid = jax.lax.axis_index("s") * _NC + jax.lax.axis_index("c")
        base = wid * bpw
        for j in range(bpw // _CH):
            off = base + j * _CH
            pltpu.sync_copy(i_hbm.at[pl.ds(off, _CH)], idx_v)
            pltpu.async_copy(tab_hbm.at[idx_v], rows_v, sem).wait()
            pltpu.sync_copy(rows_v, o_hbm.at[pl.ds(off, _CH)])

    return k(table, idx_flat)


def _dec_body(emb_ref, d1_ref, c1_ref, d2_ref, c2_ref, d3_ref, c3_ref, xr_ref):
    h = jnp.dot(emb_ref[...], d1_ref[...],
                preferred_element_type=jnp.float32) + c1_ref[...]
    h = jnp.maximum(h, 0.0)
    h = jnp.dot(h, d2_ref[...], preferred_element_type=jnp.float32) + c2_ref[...]
    h = jnp.maximum(h, 0.0)
    xr_ref[...] = (jnp.dot(h, d3_ref[...], preferred_element_type=jnp.float32)
                   + c3_ref[...])


def _decode_tc(embc, D1, c1r, D2, c2r, D3, c3r, bb):
    n, EMB = embc.shape
    IN = D3.shape[1]
    grid = (n // bb,)

    def full(a):
        return pl.BlockSpec(a.shape, lambda i: (0,) * a.ndim)

    in_specs = [pl.BlockSpec((bb, EMB), lambda i: (i, 0))] + [
        full(a) for a in (D1, c1r, D2, c2r, D3, c3r)]
    return pl.pallas_call(
        _dec_body,
        grid=grid,
        in_specs=in_specs,
        out_specs=pl.BlockSpec((bb, IN), lambda i: (i, 0)),
        out_shape=jax.ShapeDtypeStruct((n, IN), jnp.float32),
    )(embc, D1, c1r, D2, c2r, D3, c3r)


@functools.partial(jax.jit, static_argnames=("bb", "nchunk"))
def _run(x, W1, b1, W2, b2, W3, b3, Wc, D1, c1, D2, c2, D3, c3,
         bb=2048, nchunk=2):
    B = x.shape[0]
    b1r, b2r, b3r = b1[None, :], b2[None, :], b3[None, :]
    c1r, c2r, c3r = c1[None, :], c2[None, :], c3[None, :]
    table = Wc.T

    cs = B // nchunk
    zes, idxs = [], []
    for c in range(nchunk):
        ze_c, idx_c = _encode_tc(x[c * cs:(c + 1) * cs],
                                 W1, b1r, W2, b2r, W3, b3r, Wc, bb)
        zes.append(ze_c)
        idxs.append(idx_c)
    embs = [_gather_sc(table, idx_c.reshape(cs)) for idx_c in idxs]
    xrs = [_decode_tc(emb_c, D1, c1r, D2, c2r, D3, c3r, bb) for emb_c in embs]
    x_recon = jnp.concatenate(xrs, axis=0)
    z_e = jnp.concatenate(zes, axis=0)
    emb = jnp.concatenate(embs, axis=0)
    return x_recon, z_e, emb


def kernel(x, W1, b1, W2, b2, W3, b3, Wc, D1, c1, D2, c2, D3, c3):
    x_recon, z_e, emb = _run(x, W1, b1, W2, b2, W3, b3, Wc,
                             D1, c1, D2, c2, D3, c3)
    return (x_recon, z_e, emb)


# speedup vs baseline: 4.7961x; 4.7961x over previous
import functools

import jax
import jax.numpy as jnp
from jax.experimental import pallas as pl


def _fused_body(x_ref, w1_ref, b1_ref, w2_ref, b2_ref, w3_ref, b3_ref,
                wc_ref, d1_ref, c1_ref, d2_ref, c2_ref, d3_ref, c3_ref,
                xr_ref, ze_ref, emb_ref):
    bf = jnp.bfloat16
    acc = jnp.dot(x_ref[...], w1_ref[...],
                  preferred_element_type=jnp.float32) + b1_ref[...]
    h = jnp.maximum(acc.astype(bf), 0)
    acc = jnp.dot(h, w2_ref[...],
                  preferred_element_type=jnp.float32) + b2_ref[...]
    h = jnp.maximum(acc.astype(bf), 0)
    ze = jnp.dot(h, w3_ref[...],
                 preferred_element_type=jnp.float32) + b3_ref[...]
    ze_ref[...] = ze

    wc = wc_ref[...]
    cnorm = jnp.sum(wc * wc, axis=0, keepdims=True)
    wcb = wc.astype(bf)
    dist = cnorm - 2.0 * jnp.dot(ze.astype(bf), wcb,
                                 preferred_element_type=jnp.float32)
    idx = jnp.argmin(dist, axis=1)
    onehot = (jax.lax.broadcasted_iota(jnp.int32, dist.shape, 1)
              == idx[:, None]).astype(bf)
    emb = jax.lax.dot_general(onehot, wcb, (((1,), (1,)), ((), ())),
                              preferred_element_type=jnp.float32)
    emb_ref[...] = emb

    acc = jnp.dot(emb.astype(bf), d1_ref[...],
                  preferred_element_type=jnp.float32) + c1_ref[...]
    h = jnp.maximum(acc.astype(bf), 0)
    acc = jnp.dot(h, d2_ref[...],
                  preferred_element_type=jnp.float32) + c2_ref[...]
    h = jnp.maximum(acc.astype(bf), 0)
    xr_ref[...] = (jnp.dot(h, d3_ref[...], preferred_element_type=jnp.float32)
                   + c3_ref[...])


@functools.partial(jax.jit, static_argnames=("bb",))
def _run(x, W1, b1, W2, b2, W3, b3, Wc, D1, c1, D2, c2, D3, c3, bb=2048):
    B, IN = x.shape
    EMB = W3.shape[1]
    grid = (B // bb,)

    def full(a):
        return pl.BlockSpec(a.shape, lambda i: (0,) * a.ndim)

    b1r, b2r, b3r = b1[None, :], b2[None, :], b3[None, :]
    c1r, c2r, c3r = c1[None, :], c2[None, :], c3[None, :]
    bf = jnp.bfloat16
    x = x.astype(bf)
    W1, W2, W3 = W1.astype(bf), W2.astype(bf), W3.astype(bf)
    D1, D2, D3 = D1.astype(bf), D2.astype(bf), D3.astype(bf)

    out_shapes = (
        jax.ShapeDtypeStruct((B, IN), jnp.float32),
        jax.ShapeDtypeStruct((B, EMB), jnp.float32),
        jax.ShapeDtypeStruct((B, EMB), jnp.float32),
    )
    out_specs = (
        pl.BlockSpec((bb, IN), lambda i: (i, 0)),
        pl.BlockSpec((bb, EMB), lambda i: (i, 0)),
        pl.BlockSpec((bb, EMB), lambda i: (i, 0)),
    )
    in_specs = [pl.BlockSpec((bb, IN), lambda i: (i, 0))] + [
        full(a) for a in (W1, b1r, W2, b2r, W3, b3r, Wc,
                          D1, c1r, D2, c2r, D3, c3r)]
    return pl.pallas_call(
        _fused_body,
        grid=grid,
        in_specs=in_specs,
        out_specs=out_specs,
        out_shape=out_shapes,
    )(x, W1, b1r, W2, b2r, W3, b3r, Wc, D1, c1r, D2, c2r, D3, c3r)


def kernel(x, W1, b1, W2, b2, W3, b3, Wc, D1, c1, D2, c2, D3, c3):
    x_recon, z_e, emb = _run(x, W1, b1, W2, b2, W3, b3, Wc,
                             D1, c1, D2, c2, D3, c3)
    return (x_recon, z_e, emb)


# speedup vs baseline: 6.4706x; 1.3491x over previous
import functools

import jax
import jax.numpy as jnp
from jax.experimental import pallas as pl
from jax.experimental.pallas import tpu as pltpu


_NSPLIT = 2


def _fused_body(x_ref, w1_ref, b1_ref, w2_ref, b2_ref, w3_ref, b3_ref,
                wc_ref, d1_ref, c1_ref, d2_ref, c2_ref, d3_ref, c3_ref,
                xr_ref, ze_ref, emb_ref):
    wc = wc_ref[...]
    cnorm = jnp.sum(wc * wc, axis=0, keepdims=True)
    bb = x_ref.shape[0]
    sb = bb // _NSPLIT
    for s in range(_NSPLIT):
        rows = pl.ds(s * sb, sb)
        h = jnp.dot(x_ref[rows, :], w1_ref[...],
                    preferred_element_type=jnp.float32) + b1_ref[...]
        h = jnp.maximum(h, 0.0)
        h = jnp.dot(h, w2_ref[...], preferred_element_type=jnp.float32) + b2_ref[...]
        h = jnp.maximum(h, 0.0)
        ze = jnp.dot(h, w3_ref[...], preferred_element_type=jnp.float32) + b3_ref[...]
        ze_ref[rows, :] = ze

        dist = cnorm - 2.0 * jnp.dot(ze, wc, preferred_element_type=jnp.float32)
        idx = jnp.argmin(dist, axis=1)
        onehot = (jax.lax.broadcasted_iota(jnp.int32, dist.shape, 1)
                  == idx[:, None]).astype(jnp.float32)
        emb = jax.lax.dot_general(onehot, wc, (((1,), (1,)), ((), ())),
                                  preferred_element_type=jnp.float32)
        emb_ref[rows, :] = emb

        h = jnp.dot(emb, d1_ref[...], preferred_element_type=jnp.float32) + c1_ref[...]
        h = jnp.maximum(h, 0.0)
        h = jnp.dot(h, d2_ref[...], preferred_element_type=jnp.float32) + c2_ref[...]
        h = jnp.maximum(h, 0.0)
        xr_ref[rows, :] = (jnp.dot(h, d3_ref[...],
                                   preferred_element_type=jnp.float32) + c3_ref[...])


@functools.partial(jax.jit, static_argnames=("bb",))
def _run(x, W1, b1, W2, b2, W3, b3, Wc, D1, c1, D2, c2, D3, c3, bb=2048):
    B, IN = x.shape
    HID = W1.shape[1]
    HALF = W2.shape[1]
    EMB = W3.shape[1]
    K = Wc.shape[1]
    grid = (B // bb,)

    def full(a):
        return pl.BlockSpec(a.shape, lambda i: (0,) * a.ndim)

    b1r, b2r, b3r = b1[None, :], b2[None, :], b3[None, :]
    c1r, c2r, c3r = c1[None, :], c2[None, :], c3[None, :]

    batch_spec = pl.BlockSpec((bb, IN), lambda i: (i, 0))
    out_shapes = (
        jax.ShapeDtypeStruct((B, IN), jnp.float32),
        jax.ShapeDtypeStruct((B, EMB), jnp.float32),
        jax.ShapeDtypeStruct((B, EMB), jnp.float32),
    )
    out_specs = (
        pl.BlockSpec((bb, IN), lambda i: (i, 0)),
        pl.BlockSpec((bb, EMB), lambda i: (i, 0)),
        pl.BlockSpec((bb, EMB), lambda i: (i, 0)),
    )
    in_specs = [batch_spec] + [full(a) for a in
                               (W1, b1r, W2, b2r, W3, b3r, Wc,
                                D1, c1r, D2, c2r, D3, c3r)]
    return pl.pallas_call(
        _fused_body,
        grid=grid,
        in_specs=in_specs,
        out_specs=out_specs,
        out_shape=out_shapes,
        compiler_params=pltpu.CompilerParams(
            dimension_semantics=("parallel",)),
    )(x, W1, b1r, W2, b2r, W3, b3r, Wc, D1, c1r, D2, c2r, D3, c3r)


def kernel(x, W1, b1, W2, b2, W3, b3, Wc, D1, c1, D2, c2, D3, c3):
    x_recon, z_e, emb = _run(x, W1, b1, W2, b2, W3, b3, Wc,
                             D1, c1, D2, c2, D3, c3)
    return (x_recon, z_e, emb)


# speedup vs baseline: 6.6855x; 1.0332x over previous
import functools

import jax
import jax.numpy as jnp
from jax.experimental import pallas as pl
from jax.experimental.pallas import tpu as pltpu


_NSPLIT = 2


def _fused_body(x_ref, w1_ref, b1_ref, w2_ref, b2_ref, w3_ref, b3_ref,
                wc_ref, d1_ref, c1_ref, d2_ref, c2_ref, d3_ref, c3_ref,
                xr_ref, ze_ref, emb_ref):
    wc = wc_ref[...]
    cnorm = jnp.sum(wc * wc, axis=0, keepdims=True)
    wcm2 = wc * -2.0
    bb = x_ref.shape[0]
    sb = bb // _NSPLIT
    for s in range(_NSPLIT):
        rows = pl.ds(s * sb, sb)
        h = jnp.dot(x_ref[rows, :], w1_ref[...],
                    preferred_element_type=jnp.float32) + b1_ref[...]
        h = jnp.maximum(h, 0.0)
        h = jnp.dot(h, w2_ref[...], preferred_element_type=jnp.float32) + b2_ref[...]
        h = jnp.maximum(h, 0.0)
        ze = jnp.dot(h, w3_ref[...], preferred_element_type=jnp.float32) + b3_ref[...]
        ze_ref[rows, :] = ze

        dist = jnp.dot(ze, wcm2, preferred_element_type=jnp.float32) + cnorm
        idx = jnp.argmin(dist, axis=1)
        onehot = (jax.lax.broadcasted_iota(jnp.int32, dist.shape, 1)
                  == idx[:, None]).astype(jnp.float32)
        emb = jax.lax.dot_general(onehot, wc, (((1,), (1,)), ((), ())),
                                  preferred_element_type=jnp.float32)
        emb_ref[rows, :] = emb

        h = jnp.dot(emb, d1_ref[...], preferred_element_type=jnp.float32) + c1_ref[...]
        h = jnp.maximum(h, 0.0)
        h = jnp.dot(h, d2_ref[...], preferred_element_type=jnp.float32) + c2_ref[...]
        h = jnp.maximum(h, 0.0)
        xr_ref[rows, :] = (jnp.dot(h, d3_ref[...],
                                   preferred_element_type=jnp.float32) + c3_ref[...])


@functools.partial(jax.jit, static_argnames=("bb",))
def _run(x, W1, b1, W2, b2, W3, b3, Wc, D1, c1, D2, c2, D3, c3, bb=4096):
    B, IN = x.shape
    HID = W1.shape[1]
    HALF = W2.shape[1]
    EMB = W3.shape[1]
    K = Wc.shape[1]
    grid = (B // bb,)

    def full(a):
        return pl.BlockSpec(a.shape, lambda i: (0,) * a.ndim)

    b1r, b2r, b3r = b1[None, :], b2[None, :], b3[None, :]
    c1r, c2r, c3r = c1[None, :], c2[None, :], c3[None, :]

    batch_spec = pl.BlockSpec((bb, IN), lambda i: (i, 0))
    out_shapes = (
        jax.ShapeDtypeStruct((B, IN), jnp.float32),
        jax.ShapeDtypeStruct((B, EMB), jnp.float32),
        jax.ShapeDtypeStruct((B, EMB), jnp.float32),
    )
    out_specs = (
        pl.BlockSpec((bb, IN), lambda i: (i, 0)),
        pl.BlockSpec((bb, EMB), lambda i: (i, 0)),
        pl.BlockSpec((bb, EMB), lambda i: (i, 0)),
    )
    in_specs = [batch_spec] + [full(a) for a in
                               (W1, b1r, W2, b2r, W3, b3r, Wc,
                                D1, c1r, D2, c2r, D3, c3r)]
    return pl.pallas_call(
        _fused_body,
        grid=grid,
        in_specs=in_specs,
        out_specs=out_specs,
        out_shape=out_shapes,
        compiler_params=pltpu.CompilerParams(
            dimension_semantics=("parallel",)),
    )(x, W1, b1r, W2, b2r, W3, b3r, Wc, D1, c1r, D2, c2r, D3, c3r)


def kernel(x, W1, b1, W2, b2, W3, b3, Wc, D1, c1, D2, c2, D3, c3):
    x_recon, z_e, emb = _run(x, W1, b1, W2, b2, W3, b3, Wc,
                             D1, c1, D2, c2, D3, c3)
    return (x_recon, z_e, emb)
